# baseline (speedup 1.0000x reference)
import jax
import jax.numpy as jnp
from jax.experimental import pallas as pl


def _fold_kernel(m1a_ref, m2_ref, out_ref):
    out_ref[...] = jnp.dot(
        m1a_ref[...], m2_ref[...], preferred_element_type=jnp.float32
    )


def _gemm_kernel(x_ref, m_ref, b_ref, out_ref):
    mb = m_ref[...]
    for j in range(x_ref.shape[0]):
        acc = jnp.dot(x_ref[j], mb, preferred_element_type=jnp.float32)
        out_ref[j] = (acc + b_ref[...]).astype(jnp.bfloat16)


SAMPLES_PER_BLOCK = 8


def kernel(x, A, W1, b1, W2, b2):
    n, t, v, ci = x.shape
    k = A.shape[0]
    h = W1.shape[0] // k
    co = W2.shape[0] // k
    p, r = v * ci, v * co
    tp = (t + 7) // 8 * 8 + 4
    pp = 128
    rp = (r + 127) // 128 * 128

    W1r = W1.reshape(k, h, ci).transpose(0, 2, 1)
    W2r = W2.reshape(k, co, h).transpose(0, 2, 1)
    M1 = (A[:, :, None, :, None] * W1r[:, None, :, None, :]).sum(0)
    M1 = M1.reshape(p, v * h)
    M2 = (A[:, :, None, :, None] * W2r[:, None, :, None, :]).sum(0)
    M2 = M2.reshape(v * h, r)
    S = A.sum(axis=1)
    b1r = b1.reshape(k, h)
    b2r = b2.reshape(k, co)
    B1 = (S.T[:, :, None] * b1r[None, :, :]).sum(1).reshape(1, v * h)
    b2eff = (S.T[:, :, None] * b2r[None, :, :]).sum(1).reshape(1, r)

    m1a = jnp.concatenate([M1, B1], axis=0)

    mfold = pl.pallas_call(
        _fold_kernel,
        out_shape=jax.ShapeDtypeStruct((p + 1, r), jnp.float32),
    )(m1a, M2)

    M = mfold[:p]
    beff = mfold[p:] + b2eff

    X3 = x.reshape(n, t, p)
    Xp = jax.lax.pad(
        X3.astype(jnp.bfloat16), jnp.bfloat16(0),
        ((0, 0, 0), (0, tp - t, 0), (0, pp - p, 0)))
    Mp = jax.lax.pad(
        M.astype(jnp.bfloat16), jnp.bfloat16(0),
        ((0, pp - p, 0), (0, rp - r, 0)))
    bp = jax.lax.pad(beff, jnp.float32(0), ((0, 0, 0), (0, rp - r, 0)))

    bn = SAMPLES_PER_BLOCK
    padded = pl.pallas_call(
        _gemm_kernel,
        grid=(n // bn,),
        in_specs=[
            pl.BlockSpec((bn, tp, pp), lambda i: (i, 0, 0)),
            pl.BlockSpec((pp, rp), lambda i: (0, 0)),
            pl.BlockSpec((1, rp), lambda i: (0, 0)),
        ],
        out_specs=pl.BlockSpec((bn, tp, rp), lambda i: (i, 0, 0)),
        out_shape=jax.ShapeDtypeStruct((n, tp, rp), jnp.bfloat16),
    )(Xp, Mp, bp)

    return padded[:, :t, :r].astype(jnp.float32)
